# baseline (speedup 1.0000x reference)
import jax
import jax.numpy as jnp
from jax.experimental import pallas as pl
from jax.experimental.pallas import tpu as pltpu

B = 256
D = 2048
C = 8
CLS_PER_CAM = 1000
TOTAL_CLS = C * CLS_PER_CAM
NDATA = 16384
T = 0.07
HARD_NEG_K = 50
LOSS_WEIGHT = 0.5

_NEG_BIG = -1e9

_KEY16_HI = 0x4180
_KEY16_LO = -0x4180 - 2


def _key16_to_f32(k):
    b = jnp.where(k >= 0, k, k ^ jnp.int32(0x7FFF))
    return jax.lax.bitcast_convert_type(b << 16, jnp.float32)


def _slab_copy(mem_hbm, buf_ref, sem, slab, slot):
    return pltpu.make_async_copy(
        mem_hbm.at[pl.ds(slab * CLS_PER_CAM, CLS_PER_CAM), :],
        buf_ref.at[slot], sem.at[slot])


def _cap_kernel(x_ref, cams_ref, mapped_ref, mem_hbm,
                intra_ref, inter_ref,
                xn_ref, t_ref, pos_ref, buf_ref, bb_ref, sem):
    cc = pl.program_id(0)

    @pl.when(cc == 0)
    def _init():
        _slab_copy(mem_hbm, buf_ref, sem, 0, 0).start()
        _slab_copy(mem_hbm, buf_ref, sem, 1, 1).start()
        x = x_ref[...]
        inv = jax.lax.rsqrt(jnp.sum(x * x, axis=1, keepdims=True))
        xn_ref[...] = (x * (inv * (1.0 / T))).astype(jnp.bfloat16)
        pos_ref[...] = jnp.zeros((B, 1), jnp.float32)

    def _cast_stage(slot):
        _slab_copy(mem_hbm, buf_ref, sem, cc, slot).wait()
        bb_ref[slot] = buf_ref[slot].astype(jnp.bfloat16)
        @pl.when(cc + 2 < C)
        def _():
            _slab_copy(mem_hbm, buf_ref, sem, cc + 2, slot).start()

    @pl.when(jnp.logical_and(cc < C, jax.lax.rem(cc, 2) == 0))
    def _cast_even():
        _cast_stage(0)

    @pl.when(jnp.logical_and(cc < C, jax.lax.rem(cc, 2) == 1))
    def _cast_odd():
        _cast_stage(1)

    def _slab_compute(prev, slot):
        xn = xn_ref[...]
        t = jax.lax.dot_general(
            xn, bb_ref[slot], (((1,), (1,)), ((), ())),
            preferred_element_type=jnp.float32)
        cams = cams_ref[...]
        mapped = mapped_ref[...]
        row_in_cam = cams == prev
        col = jax.lax.broadcasted_iota(jnp.int32, (B, CLS_PER_CAM), 1)
        pos_mask = row_in_cam & (col == mapped)
        pos = jnp.sum(jnp.where(pos_mask, t, 0.0), axis=1, keepdims=True)
        pos_ref[...] = jnp.where(row_in_cam, pos, pos_ref[...])
        t_masked = jnp.where(pos_mask, _NEG_BIG, t)
        for k in range(C):
            @pl.when(prev == k)
            def _(k=k):
                t_ref[k] = t_masked

    @pl.when(jnp.logical_and(cc >= 1, jax.lax.rem(cc, 2) == 1))
    def _mm_even():
        _slab_compute(cc - 1, 0)

    @pl.when(jnp.logical_and(cc >= 1, jax.lax.rem(cc, 2) == 0))
    def _mm_odd():
        _slab_compute(cc - 1, 1)

    @pl.when(cc == C)
    def _select_and_reduce():
        t = t_ref[...]
        pos = pos_ref[...]
        lo = jnp.full((1, B, 1), _KEY16_LO, jnp.int32)
        hi = jnp.full((1, B, 1), _KEY16_HI, jnp.int32)
        clo = jnp.full((1, B, 1), float(TOTAL_CLS - 1), jnp.float32)
        chi = jnp.zeros((1, B, 1), jnp.float32)

        def body(_, carry):
            lo, hi, clo, chi = carry
            mid = (lo + hi) >> 1
            thr = _key16_to_f32(mid)
            cnt = jnp.sum(jnp.where(t > thr, 1.0, 0.0), axis=(0, 2),
                          keepdims=True)
            ge = cnt >= jnp.float32(HARD_NEG_K)
            return (jnp.where(ge, mid, lo), jnp.where(ge, hi, mid),
                    jnp.where(ge, cnt, clo), jnp.where(ge, chi, cnt))

        lo, hi, cnt_ge, cnt_gt = jax.lax.fori_loop(
            0, 16, body, (lo, hi, clo, chi))
        tau = _key16_to_f32(hi)
        tau_lo = _key16_to_f32(lo)
        pos3 = pos.reshape(1, B, 1)
        mref = jnp.maximum(tau, pos3)
        e = jnp.exp(t - mref)
        s_top = jnp.sum(jnp.where(t > tau, e, 0.0), axis=(0, 2),
                        keepdims=True)
        s_ge = jnp.sum(jnp.where(t > tau_lo, e, 0.0), axis=(0, 2),
                       keepdims=True)
        slab_sum = jnp.sum(e, axis=2, keepdims=True)
        cnt_eq = cnt_ge - cnt_gt
        s_fill = ((jnp.float32(HARD_NEG_K) - cnt_gt)
                  * (s_ge - s_top) / cnt_eq)
        e_pos = jnp.exp(pos3 - mref)
        b_inter = (jnp.log(s_top + s_fill + e_pos)
                   + mref - pos3).reshape(B, 1)

        cams = cams_ref[...]
        own_sum = jnp.zeros((1, B, 1), jnp.float32)
        for k in range(C):
            own_sum = own_sum + jnp.where(cams.reshape(1, B, 1) == k,
                                          slab_sum[k][None], 0.0)
        a_intra = (jnp.log(own_sum + e_pos)
                   + mref - pos3).reshape(B, 1)

        li = jnp.zeros((1, 1), jnp.float32)
        le = jnp.zeros((1, 1), jnp.float32)
        for k in range(C):
            mask = cams == k
            n = jnp.sum(mask.astype(jnp.float32), axis=(0, 1), keepdims=True)
            denom = jnp.maximum(n, 1.0)
            sa = jnp.sum(jnp.where(mask, a_intra, 0.0), axis=(0, 1),
                         keepdims=True)
            sb = jnp.sum(jnp.where(mask, b_inter, 0.0), axis=(0, 1),
                         keepdims=True)
            present = n > 0.0
            li = li + jnp.where(present, sa / denom, 0.0)
            le = le + jnp.where(present, sb / denom, 0.0)
        intra_ref[...] = li
        inter_ref[...] = jnp.float32(LOSS_WEIGHT) * le


def _cap_pallas(inputs, cams, mapped, memory, interpret=False):
    return pl.pallas_call(
        _cap_kernel,
        grid=(C + 1,),
        in_specs=[
            pl.BlockSpec((B, D), lambda i: (0, 0)),
            pl.BlockSpec((B, 1), lambda i: (0, 0)),
            pl.BlockSpec((B, 1), lambda i: (0, 0)),
            pl.BlockSpec(memory_space=pltpu.MemorySpace.HBM),
        ],
        out_specs=[
            pl.BlockSpec((1, 1), lambda i: (0, 0)),
            pl.BlockSpec((1, 1), lambda i: (0, 0)),
        ],
        out_shape=[
            jax.ShapeDtypeStruct((1, 1), jnp.float32),
            jax.ShapeDtypeStruct((1, 1), jnp.float32),
        ],
        scratch_shapes=[
            pltpu.VMEM((B, D), jnp.bfloat16),
            pltpu.VMEM((C, B, CLS_PER_CAM), jnp.float32),
            pltpu.VMEM((B, 1), jnp.float32),
            pltpu.VMEM((2, CLS_PER_CAM, D), jnp.float32),
            pltpu.VMEM((2, CLS_PER_CAM, D), jnp.bfloat16),
            pltpu.SemaphoreType.DMA((2,)),
        ],
        interpret=interpret,
    )(inputs, cams, mapped, memory)


@jax.jit
def kernel(inputs, indexes, labels, memory):
    batch_labels = labels[indexes]
    cams = (batch_labels // CLS_PER_CAM).astype(jnp.int32).reshape(B, 1)
    mapped = (batch_labels % CLS_PER_CAM).astype(jnp.int32).reshape(B, 1)
    out = _cap_pallas(inputs, cams, mapped, memory)
    return (out[0][0, 0], out[1][0, 0])


# speedup vs baseline: 1.0377x; 1.0377x over previous
import jax
import jax.numpy as jnp
from jax.experimental import pallas as pl
from jax.experimental.pallas import tpu as pltpu

B = 256
D = 2048
C = 8
CLS_PER_CAM = 1000
TOTAL_CLS = C * CLS_PER_CAM
NDATA = 16384
T = 0.07
HARD_NEG_K = 50
LOSS_WEIGHT = 0.5

_PAD = 1024
_W = C * _PAD

_NEG_BIG = -1e9

_KEY16_HI = 0x4180
_KEY16_LO = -0x4180 - 2


def _key16_to_f32(k):
    b = jnp.where(k >= 0, k, k ^ jnp.int32(0x7FFF))
    return jax.lax.bitcast_convert_type(b << 16, jnp.float32)


def _slab_copy(mem_hbm, buf_ref, sem, slab, slot):
    return pltpu.make_async_copy(
        mem_hbm.at[pl.ds(slab * CLS_PER_CAM, CLS_PER_CAM), :],
        buf_ref.at[slot], sem.at[slot])


def _cap_kernel(x_ref, cams_ref, mapped_ref, mem_hbm,
                intra_ref, inter_ref,
                xn_ref, t_ref, pos_ref, buf_ref, sem):
    cc = pl.program_id(0)

    @pl.when(cc == 0)
    def _init():
        _slab_copy(mem_hbm, buf_ref, sem, 0, 0).start()
        _slab_copy(mem_hbm, buf_ref, sem, 1, 1).start()
        x = x_ref[...]
        inv = jax.lax.rsqrt(jnp.sum(x * x, axis=1, keepdims=True))
        xn_ref[...] = (x * (inv * (1.0 / T))).astype(jnp.bfloat16)
        pos_ref[...] = jnp.zeros((B, 1), jnp.float32)
        t_ref[...] = jnp.full((B, _W), _NEG_BIG, jnp.float32)

    def _slab_compute(slot):
        _slab_copy(mem_hbm, buf_ref, sem, cc, slot).wait()
        xn = xn_ref[...]
        blk = buf_ref[slot].astype(jnp.bfloat16)
        t = jax.lax.dot_general(
            xn, blk, (((1,), (1,)), ((), ())),
            preferred_element_type=jnp.float32)
        cams = cams_ref[...]
        mapped = mapped_ref[...]
        row_in_cam = cams == cc
        col = jax.lax.broadcasted_iota(jnp.int32, (B, CLS_PER_CAM), 1)
        pos_mask = row_in_cam & (col == mapped)
        pos = jnp.sum(jnp.where(pos_mask, t, 0.0), axis=1, keepdims=True)
        pos_ref[...] = jnp.where(row_in_cam, pos, pos_ref[...])
        t_masked = jnp.where(pos_mask, _NEG_BIG, t)
        for k in range(C):
            @pl.when(cc == k)
            def _(k=k):
                t_ref[:, k * _PAD:k * _PAD + CLS_PER_CAM] = t_masked
        @pl.when(cc + 2 < C)
        def _():
            _slab_copy(mem_hbm, buf_ref, sem, cc + 2, slot).start()

    @pl.when(jnp.logical_and(cc < C, jax.lax.rem(cc, 2) == 0))
    def _even():
        _slab_compute(0)

    @pl.when(jnp.logical_and(cc < C, jax.lax.rem(cc, 2) == 1))
    def _odd():
        _slab_compute(1)

    @pl.when(cc == C)
    def _select_and_reduce():
        t = t_ref[...]
        pos = pos_ref[...]
        lo = jnp.full((B, 1), _KEY16_LO, jnp.int32)
        hi = jnp.full((B, 1), _KEY16_HI, jnp.int32)
        clo = jnp.full((B, 1), float(TOTAL_CLS - 1), jnp.float32)
        chi = jnp.zeros((B, 1), jnp.float32)

        def body(_, carry):
            lo, hi, clo, chi = carry
            mid = (lo + hi) >> 1
            thr = _key16_to_f32(mid)
            cnt = jnp.sum(jnp.where(t > thr, 1.0, 0.0), axis=1,
                          keepdims=True)
            ge = cnt >= jnp.float32(HARD_NEG_K)
            return (jnp.where(ge, mid, lo), jnp.where(ge, hi, mid),
                    jnp.where(ge, cnt, clo), jnp.where(ge, chi, cnt))

        lo, hi, cnt_ge, cnt_gt = jax.lax.fori_loop(
            0, 16, body, (lo, hi, clo, chi))
        tau = _key16_to_f32(hi)
        tau_lo = _key16_to_f32(lo)
        mref = jnp.maximum(tau, pos)
        e = jnp.exp(t - mref)
        s_top = jnp.sum(jnp.where(t > tau, e, 0.0), axis=1, keepdims=True)
        s_ge = jnp.sum(jnp.where(t > tau_lo, e, 0.0), axis=1, keepdims=True)
        cnt_eq = cnt_ge - cnt_gt
        s_fill = ((jnp.float32(HARD_NEG_K) - cnt_gt)
                  * (s_ge - s_top) / cnt_eq)
        e_pos = jnp.exp(pos - mref)
        b_inter = jnp.log(s_top + s_fill + e_pos) + mref - pos

        cams = cams_ref[...]
        own_sum = jnp.zeros((B, 1), jnp.float32)
        for k in range(C):
            sk = jnp.sum(e[:, k * _PAD:(k + 1) * _PAD], axis=1,
                         keepdims=True)
            own_sum = own_sum + jnp.where(cams == k, sk, 0.0)
        a_intra = jnp.log(own_sum + e_pos) + mref - pos

        li = jnp.zeros((1, 1), jnp.float32)
        le = jnp.zeros((1, 1), jnp.float32)
        for k in range(C):
            mask = cams == k
            n = jnp.sum(mask.astype(jnp.float32), axis=(0, 1), keepdims=True)
            denom = jnp.maximum(n, 1.0)
            sa = jnp.sum(jnp.where(mask, a_intra, 0.0), axis=(0, 1),
                         keepdims=True)
            sb = jnp.sum(jnp.where(mask, b_inter, 0.0), axis=(0, 1),
                         keepdims=True)
            present = n > 0.0
            li = li + jnp.where(present, sa / denom, 0.0)
            le = le + jnp.where(present, sb / denom, 0.0)
        intra_ref[...] = li
        inter_ref[...] = jnp.float32(LOSS_WEIGHT) * le


def _cap_pallas(inputs, cams, mapped, memory, interpret=False):
    return pl.pallas_call(
        _cap_kernel,
        grid=(C + 1,),
        in_specs=[
            pl.BlockSpec((B, D), lambda i: (0, 0)),
            pl.BlockSpec((B, 1), lambda i: (0, 0)),
            pl.BlockSpec((B, 1), lambda i: (0, 0)),
            pl.BlockSpec(memory_space=pltpu.MemorySpace.HBM),
        ],
        out_specs=[
            pl.BlockSpec((1, 1), lambda i: (0, 0)),
            pl.BlockSpec((1, 1), lambda i: (0, 0)),
        ],
        out_shape=[
            jax.ShapeDtypeStruct((1, 1), jnp.float32),
            jax.ShapeDtypeStruct((1, 1), jnp.float32),
        ],
        scratch_shapes=[
            pltpu.VMEM((B, D), jnp.bfloat16),
            pltpu.VMEM((B, _W), jnp.float32),
            pltpu.VMEM((B, 1), jnp.float32),
            pltpu.VMEM((2, CLS_PER_CAM, D), jnp.float32),
            pltpu.SemaphoreType.DMA((2,)),
        ],
        interpret=interpret,
    )(inputs, cams, mapped, memory)


@jax.jit
def kernel(inputs, indexes, labels, memory):
    batch_labels = labels[indexes]
    cams = (batch_labels // CLS_PER_CAM).astype(jnp.int32).reshape(B, 1)
    mapped = (batch_labels % CLS_PER_CAM).astype(jnp.int32).reshape(B, 1)
    out = _cap_pallas(inputs, cams, mapped, memory)
    return (out[0][0, 0], out[1][0, 0])
